# Initial kernel scaffold; baseline (speedup 1.0000x reference)
import jax
import jax.numpy as jnp
from jax.experimental import pallas as pl


def kernel(patient_ids, disease_pos_ids, disease_neg_ids, ent_embed, edge_row, edge_col, edge_val, W1_0, b1_0, W2_0, b2_0, W1_1, b1_1, W2_1, b2_1):
    raise NotImplementedError("write your pallas kernel here")


# speedup vs baseline: 4.3667x; 4.3667x over previous
import functools

import jax
import jax.numpy as jnp
from jax import lax
from jax.experimental import pallas as pl
from jax.experimental.pallas import tpu as pltpu
from jax.experimental.pallas import tpu_sc as plsc

N = 10000
NP = 10240
E = 320000
B = 4096

NC = 2
NS = 16
NW = NC * NS
L = 16

K = 128
E_PAD = ((E + NW * K - 1) // (NW * K)) * (NW * K)
EC = E_PAD // NW
NCHUNK = EC // K
RPT = NP // NS

_mesh = lambda: plsc.VectorSubcoreMesh(
    core_axis_name="c", subcore_axis_name="s", num_cores=NC, num_subcores=NS)


def _make_spmv(D):
  DL = D // L

  def body(ego, ecol, erow, eval_, out, colv, rowv, valv, rows, acc, sem):
    c = lax.axis_index("c")
    s = lax.axis_index("s")
    wid = s * NC + c

    def zrow(i, _):
      for d in range(DL):
        rows[i, pl.ds(d * L, L)] = jnp.zeros((L,), jnp.float32)
      return _
    lax.fori_loop(0, K, zrow, 0)
    for t in range(RPT // K):
      pltpu.sync_copy(rows, acc.at[pl.ds(s * RPT + t * K, K)])
    plsc.subcore_barrier()

    ebase = wid * EC

    def chunk(ci, _):
      base = ebase + ci * K
      pltpu.sync_copy(ecol.at[pl.ds(base, K)], colv)
      pltpu.sync_copy(erow.at[pl.ds(base, K)], rowv)
      pltpu.sync_copy(eval_.at[pl.ds(base, K)], valv)
      pltpu.async_copy(ego.at[colv], rows, sem).wait()

      def scale(g, _):
        v16 = valv[pl.ds(g * L, L)]
        for jj in range(L):
          v = v16[jj]
          j = g * L + jj
          for d in range(DL):
            rows[j, pl.ds(d * L, L)] = rows[j, pl.ds(d * L, L)] * v
        return _
      lax.fori_loop(0, K // L, scale, 0)
      pltpu.sync_copy(rows, acc.at[rowv], add=True)
      return _
    lax.fori_loop(0, NCHUNK, chunk, 0)

    plsc.subcore_barrier()
    pltpu.sync_copy(acc.at[pl.ds(s * RPT, RPT)],
                    out.at[c, pl.ds(s * RPT, RPT)])

  return pl.kernel(
      body,
      out_type=jax.ShapeDtypeStruct((NC, NP, D), jnp.float32),
      mesh=_mesh(),
      scratch_types=[
          pltpu.VMEM((K,), jnp.int32),
          pltpu.VMEM((K,), jnp.int32),
          pltpu.VMEM((K,), jnp.float32),
          pltpu.VMEM((K, D), jnp.float32),
          pltpu.VMEM_SHARED((NP, D), jnp.float32),
          pltpu.SemaphoreType.DMA,
      ],
  )


def _dense_layer(e, parts, W1, b1, W2, b2):
  Din, Dout = W1.shape

  def body(e_ref, p_ref, w1_ref, b1_ref, w2_ref, b2_ref, eo_ref, no_ref):
    side = p_ref[0, :N, :Din] + p_ref[1, :N, :Din]
    ego = e_ref[:, :Din]
    su = ego + side
    bi = ego * side
    s1 = jnp.dot(su, w1_ref[...], preferred_element_type=jnp.float32) + b1_ref[...]
    s1 = jnp.where(s1 >= 0, s1, 0.01 * s1)
    s2 = jnp.dot(bi, w2_ref[...], preferred_element_type=jnp.float32) + b2_ref[...]
    s2 = jnp.where(s2 >= 0, s2, 0.01 * s2)
    o = s1 + s2
    zpad = jnp.zeros((N, 128 - Dout), jnp.float32)
    eo_ref[...] = jnp.concatenate([o, zpad], axis=1)
    nrm = jnp.sqrt(jnp.sum(o * o, axis=1, keepdims=True))
    no_ref[...] = jnp.concatenate([o / jnp.maximum(nrm, 1e-12), zpad], axis=1)

  return pl.pallas_call(
      body,
      out_shape=(jax.ShapeDtypeStruct((N, 128), jnp.float32),
                 jax.ShapeDtypeStruct((N, 128), jnp.float32)),
  )(e, parts, W1, b1.reshape(1, -1), W2, b2.reshape(1, -1))


BT = B // NW


def _make_score():
  dims = (128, 64, 32)
  H = 2
  BH = BT // H

  def body(e0, n1, n2, pid, qid, nid, out,
           pi, qi, ni,
           p0, q0, g0, p1, q1, g1, p2, q2, g2,
           pv, nv, sp, sq, sn, sem):
    c = lax.axis_index("c")
    s = lax.axis_index("s")
    wid = s * NC + c
    base = wid * BT

    bufs = ((p0, q0, g0, dims[0] // L),
            (p1, q1, g1, dims[1] // L),
            (p2, q2, g2, dims[2] // L))
    z = jnp.zeros((L,), jnp.float32)

    for h in range(H):
      hb = base + h * BH
      pltpu.sync_copy(pid.at[pl.ds(hb, BH)], pi)
      pltpu.sync_copy(qid.at[pl.ds(hb, BH)], qi)
      pltpu.sync_copy(nid.at[pl.ds(hb, BH)], ni)
      pltpu.async_copy(e0.at[pi], p0, sem).wait()
      pltpu.async_copy(e0.at[qi], q0, sem).wait()
      pltpu.async_copy(e0.at[ni], g0, sem).wait()
      pltpu.async_copy(n1.at[pi], p1, sem).wait()
      pltpu.async_copy(n1.at[qi], q1, sem).wait()
      pltpu.async_copy(n1.at[ni], g1, sem).wait()
      pltpu.async_copy(n2.at[pi], p2, sem).wait()
      pltpu.async_copy(n2.at[qi], q2, sem).wait()
      pltpu.async_copy(n2.at[ni], g2, sem).wait()

      def elem(j, _):
        ap = z; an = z; lp = z; lq = z; ln_ = z
        for (bp, bq, bn, dl) in bufs:
          for d in range(dl):
            vp = bp[j, pl.ds(d * L, L)]
            vq = bq[j, pl.ds(d * L, L)]
            vn = bn[j, pl.ds(d * L, L)]
            ap = ap + vp * vq
            an = an + vp * vn
            lp = lp + vp * vp
            lq = lq + vq * vq
            ln_ = ln_ + vn * vn
        o = (h * BH + j) * L
        pv[pl.ds(o, L)] = ap
        nv[pl.ds(o, L)] = an
        sp[pl.ds(o, L)] = lp
        sq[pl.ds(o, L)] = lq
        sn[pl.ds(o, L)] = ln_
        return _
      lax.fori_loop(0, BH, elem, 0)

    pltpu.sync_copy(pv, out.at[pl.ds((0 * B + base) * L, BT * L)])
    pltpu.sync_copy(nv, out.at[pl.ds((1 * B + base) * L, BT * L)])
    pltpu.sync_copy(sp, out.at[pl.ds((2 * B + base) * L, BT * L)])
    pltpu.sync_copy(sq, out.at[pl.ds((3 * B + base) * L, BT * L)])
    pltpu.sync_copy(sn, out.at[pl.ds((4 * B + base) * L, BT * L)])

  scratch = [pltpu.VMEM((BH,), jnp.int32) for _ in range(3)]
  for d in dims:
    scratch += [pltpu.VMEM((BH, 128), jnp.float32) for _ in range(3)]
  scratch += [pltpu.VMEM((BT * L,), jnp.float32) for _ in range(5)]
  scratch += [pltpu.SemaphoreType.DMA]

  return pl.kernel(
      body,
      out_type=jax.ShapeDtypeStruct((5 * B * L,), jnp.float32),
      mesh=_mesh(),
      scratch_types=scratch,
  )


def _epilogue(scores):
  def body(s_ref, o_ref):
    sc = jnp.sum(s_ref[...], axis=2)
    d = sc[0] - sc[1]
    cf = jnp.mean(jnp.maximum(-d, 0.0) + jnp.log1p(jnp.exp(-jnp.abs(d))))
    l2 = (jnp.mean(sc[2]) + jnp.mean(sc[3]) + jnp.mean(sc[4])) * 0.5
    o_ref[0, 0] = cf + 1e-05 * l2

  out = pl.pallas_call(
      body,
      out_shape=jax.ShapeDtypeStruct((1, 1), jnp.float32),
      out_specs=pl.BlockSpec(memory_space=pltpu.SMEM),
  )(scores)
  return out[0, 0]


def kernel(patient_ids, disease_pos_ids, disease_neg_ids, ent_embed,
           edge_row, edge_col, edge_val,
           W1_0, b1_0, W2_0, b2_0, W1_1, b1_1, W2_1, b2_1):
  pad = E_PAD - E
  pad_idx = jnp.arange(pad, dtype=jnp.int32) % N
  ecol = jnp.concatenate([edge_col.astype(jnp.int32), pad_idx])
  erow = jnp.concatenate([edge_row.astype(jnp.int32), pad_idx])
  evalp = jnp.concatenate([edge_val, jnp.zeros((pad,), jnp.float32)])

  parts0 = _make_spmv(128)(ent_embed, ecol, erow, evalp)
  e1, n1 = _dense_layer(ent_embed, parts0, W1_0, b1_0, W2_0, b2_0)
  parts1 = _make_spmv(128)(e1, ecol, erow, evalp)
  _, n2 = _dense_layer(e1, parts1, W1_1, b1_1, W2_1, b2_1)

  scores_flat = _make_score()(ent_embed, n1, n2,
                         patient_ids.astype(jnp.int32),
                         disease_pos_ids.astype(jnp.int32),
                         disease_neg_ids.astype(jnp.int32))
  return _epilogue(scores_flat.reshape(5, B, L))


# speedup vs baseline: 9.1239x; 2.0894x over previous
import functools

import jax
import jax.numpy as jnp
from jax import lax
from jax.experimental import pallas as pl
from jax.experimental.pallas import tpu as pltpu
from jax.experimental.pallas import tpu_sc as plsc

N = 10000
NP = 10240
E = 320000
B = 4096

NC = 2
NS = 16
NW = NC * NS
L = 16

K = 128
R = 4
E_PAD = ((E + NW * K * R - 1) // (NW * K * R)) * (NW * K * R)
EC = E_PAD // NW
NCHUNK = EC // K
RPT = NP // NS

_mesh = lambda: plsc.VectorSubcoreMesh(
    core_axis_name="c", subcore_axis_name="s", num_cores=NC, num_subcores=NS)


def _make_spmv(D):
  DL = D // L
  Q = 4

  def body(ego, ecol2, erow2, eval_, out,
           colv0, colv1, colv2, colv3, rowv0, rowv1, rowv2, rowv3,
           valv, rows0, rows1, acc,
           vsem, ls0, ls1, ls2, ls3, gs0, gs1):
    c = lax.axis_index("c")
    s = lax.axis_index("s")
    wid = s * NC + c
    rows = (rows0, rows1)
    colv = (colv0, colv1, colv2, colv3)
    rowv = (rowv0, rowv1, rowv2, rowv3)
    lsem = (ls0, ls1, ls2, ls3)
    gsem = (gs0, gs1)

    def zrow(i, _):
      for d in range(DL):
        rows0[i, pl.ds(d * L, L)] = jnp.zeros((L,), jnp.float32)
      return _
    lax.fori_loop(0, K, zrow, 0)
    for t in range(RPT // K):
      pltpu.sync_copy(rows0, acc.at[pl.ds(s * RPT + t * K, K)])

    vb = pltpu.async_copy(eval_.at[pl.ds(wid * EC, EC)], valv, vsem)
    vb.wait()
    plsc.subcore_barrier()

    ebase0 = wid * EC

    def iload(ci, q):
      pltpu.async_copy(ecol2.at[pl.ds(ebase0 + ci * K, K)], colv[q], lsem[q])
      pltpu.async_copy(erow2.at[pl.ds(ebase0 + ci * K, K)], rowv[q], lsem[q])

    def iwait(ci, q):
      pltpu.make_async_copy(ecol2.at[pl.ds(ebase0 + ci * K, K)], colv[q], lsem[q]).wait()
      pltpu.make_async_copy(erow2.at[pl.ds(ebase0 + ci * K, K)], rowv[q], lsem[q]).wait()

    def gather(ci, b, q):
      pltpu.async_copy(ego.at[colv[q]], rows[b], gsem[b])

    def gwait(ci, b, q):
      pltpu.make_async_copy(ego.at[colv[q]], rows[b], gsem[b]).wait()

    def scale(ci, b):
      ebase = ci * K
      def grp(g, _):
        v16 = valv[pl.ds(ebase + g * L, L)]
        for jj in range(L):
          v = v16[jj]
          j = g * L + jj
          for d in range(DL):
            rows[b][j, pl.ds(d * L, L)] = rows[b][j, pl.ds(d * L, L)] * v
        return _
      lax.fori_loop(0, K // L, grp, 0)

    for ci in range(3):
      iload(ci, ci)
    for ci in range(2):
      iwait(ci, ci)
      gather(ci, ci % 2, ci)

    def step(ci, u, do_gather, do_iload):
      b = u % 2
      gwait(ci, b, u)
      scale(ci, b)
      pltpu.sync_copy(rows[b], acc.at[rowv[u]], add=True)
      if do_gather:
        iwait(ci + 2, (u + 2) % Q)
        gather(ci + 2, b, (u + 2) % Q)
      if do_iload:
        iload(ci + 3, (u + 3) % Q)

    def grp4(i, _):
      for u in range(Q):
        ci = i * Q + u
        step(ci, u, True, True)
      return _
    lax.fori_loop(0, NCHUNK // Q - 1, grp4, 0)

    for u in range(Q):
      ci = NCHUNK - Q + u
      step(ci, u, ci + 2 < NCHUNK, ci + 3 < NCHUNK)

    plsc.subcore_barrier()
    pltpu.sync_copy(acc.at[pl.ds(s * RPT, RPT)],
                    out.at[c, pl.ds(s * RPT, RPT)])

  return pl.kernel(
      body,
      out_type=jax.ShapeDtypeStruct((NC, NP, D), jnp.float32),
      mesh=_mesh(),
      scratch_types=[pltpu.VMEM((K,), jnp.int32) for _ in range(8)] + [
          pltpu.VMEM((EC,), jnp.float32),
          pltpu.VMEM((K, D), jnp.float32),
          pltpu.VMEM((K, D), jnp.float32),
          pltpu.VMEM_SHARED((NP, D), jnp.float32),
      ] + [pltpu.SemaphoreType.DMA] * 7,
  )


def _dense_layer(e, parts, W1, b1, W2, b2):
  Din, Dout = W1.shape

  def body(e_ref, p_ref, w1_ref, b1_ref, w2_ref, b2_ref, eo_ref, no_ref):
    side = p_ref[0, :N, :Din] + p_ref[1, :N, :Din]
    ego = e_ref[:, :Din]
    su = ego + side
    bi = ego * side
    s1 = jnp.dot(su, w1_ref[...], preferred_element_type=jnp.float32) + b1_ref[...]
    s1 = jnp.where(s1 >= 0, s1, 0.01 * s1)
    s2 = jnp.dot(bi, w2_ref[...], preferred_element_type=jnp.float32) + b2_ref[...]
    s2 = jnp.where(s2 >= 0, s2, 0.01 * s2)
    o = s1 + s2
    zpad = jnp.zeros((N, 128 - Dout), jnp.float32)
    eo_ref[...] = jnp.concatenate([o, zpad], axis=1)
    nrm = jnp.sqrt(jnp.sum(o * o, axis=1, keepdims=True))
    no_ref[...] = jnp.concatenate([o / jnp.maximum(nrm, 1e-12), zpad], axis=1)

  return pl.pallas_call(
      body,
      out_shape=(jax.ShapeDtypeStruct((N, 128), jnp.float32),
                 jax.ShapeDtypeStruct((N, 128), jnp.float32)),
  )(e, parts, W1, b1.reshape(1, -1), W2, b2.reshape(1, -1))


BT = B // NW


def _make_score():
  dims = (128, 64, 32)
  H = 2
  BH = BT // H

  def body(e0, n1, n2, pid, qid, nid, out,
           pi, qi, ni,
           p0, q0, g0, p1, q1, g1, p2, q2, g2,
           pv, nv, sp, sq, sn, sem):
    c = lax.axis_index("c")
    s = lax.axis_index("s")
    wid = s * NC + c
    base = wid * BT

    bufs = ((p0, q0, g0, dims[0] // L),
            (p1, q1, g1, dims[1] // L),
            (p2, q2, g2, dims[2] // L))
    z = jnp.zeros((L,), jnp.float32)

    for h in range(H):
      hb = base + h * BH
      pltpu.sync_copy(pid.at[pl.ds(hb, BH)], pi)
      pltpu.sync_copy(qid.at[pl.ds(hb, BH)], qi)
      pltpu.sync_copy(nid.at[pl.ds(hb, BH)], ni)
      pltpu.async_copy(e0.at[pi], p0, sem).wait()
      pltpu.async_copy(e0.at[qi], q0, sem).wait()
      pltpu.async_copy(e0.at[ni], g0, sem).wait()
      pltpu.async_copy(n1.at[pi], p1, sem).wait()
      pltpu.async_copy(n1.at[qi], q1, sem).wait()
      pltpu.async_copy(n1.at[ni], g1, sem).wait()
      pltpu.async_copy(n2.at[pi], p2, sem).wait()
      pltpu.async_copy(n2.at[qi], q2, sem).wait()
      pltpu.async_copy(n2.at[ni], g2, sem).wait()

      def elem(j, _):
        ap = z; an = z; lp = z; lq = z; ln_ = z
        for (bp, bq, bn, dl) in bufs:
          for d in range(dl):
            vp = bp[j, pl.ds(d * L, L)]
            vq = bq[j, pl.ds(d * L, L)]
            vn = bn[j, pl.ds(d * L, L)]
            ap = ap + vp * vq
            an = an + vp * vn
            lp = lp + vp * vp
            lq = lq + vq * vq
            ln_ = ln_ + vn * vn
        o = (h * BH + j) * L
        pv[pl.ds(o, L)] = ap
        nv[pl.ds(o, L)] = an
        sp[pl.ds(o, L)] = lp
        sq[pl.ds(o, L)] = lq
        sn[pl.ds(o, L)] = ln_
        return _
      lax.fori_loop(0, BH, elem, 0)

    pltpu.sync_copy(pv, out.at[pl.ds((0 * B + base) * L, BT * L)])
    pltpu.sync_copy(nv, out.at[pl.ds((1 * B + base) * L, BT * L)])
    pltpu.sync_copy(sp, out.at[pl.ds((2 * B + base) * L, BT * L)])
    pltpu.sync_copy(sq, out.at[pl.ds((3 * B + base) * L, BT * L)])
    pltpu.sync_copy(sn, out.at[pl.ds((4 * B + base) * L, BT * L)])

  scratch = [pltpu.VMEM((BH,), jnp.int32) for _ in range(3)]
  for d in dims:
    scratch += [pltpu.VMEM((BH, 128), jnp.float32) for _ in range(3)]
  scratch += [pltpu.VMEM((BT * L,), jnp.float32) for _ in range(5)]
  scratch += [pltpu.SemaphoreType.DMA]

  return pl.kernel(
      body,
      out_type=jax.ShapeDtypeStruct((5 * B * L,), jnp.float32),
      mesh=_mesh(),
      scratch_types=scratch,
  )


def _epilogue(scores):
  def body(s_ref, o_ref):
    sc = jnp.sum(s_ref[...], axis=2)
    d = sc[0] - sc[1]
    cf = jnp.mean(jnp.maximum(-d, 0.0) + jnp.log1p(jnp.exp(-jnp.abs(d))))
    l2 = (jnp.mean(sc[2]) + jnp.mean(sc[3]) + jnp.mean(sc[4])) * 0.5
    o_ref[0, 0] = cf + 1e-05 * l2

  out = pl.pallas_call(
      body,
      out_shape=jax.ShapeDtypeStruct((1, 1), jnp.float32),
      out_specs=pl.BlockSpec(memory_space=pltpu.SMEM),
  )(scores)
  return out[0, 0]


def kernel(patient_ids, disease_pos_ids, disease_neg_ids, ent_embed,
           edge_row, edge_col, edge_val,
           W1_0, b1_0, W2_0, b2_0, W1_1, b1_1, W2_1, b2_1):
  pad = E_PAD - E
  pad_idx = jnp.arange(pad, dtype=jnp.int32) % N
  ecol = jnp.concatenate([edge_col.astype(jnp.int32), pad_idx])
  erow = jnp.concatenate([edge_row.astype(jnp.int32), pad_idx])
  evalp = jnp.concatenate([edge_val, jnp.zeros((pad,), jnp.float32)])

  parts0 = _make_spmv(128)(ent_embed, ecol, erow, evalp)
  e1, n1 = _dense_layer(ent_embed, parts0, W1_0, b1_0, W2_0, b2_0)
  parts1 = _make_spmv(128)(e1, ecol, erow, evalp)
  _, n2 = _dense_layer(e1, parts1, W1_1, b1_1, W2_1, b2_1)

  scores_flat = _make_score()(ent_embed, n1, n2,
                         patient_ids.astype(jnp.int32),
                         disease_pos_ids.astype(jnp.int32),
                         disease_neg_ids.astype(jnp.int32))
  return _epilogue(scores_flat.reshape(5, B, L))
